# baseline (speedup 1.0000x reference)
import functools

import jax
import jax.numpy as jnp
from jax import lax
from jax.experimental import pallas as pl
from jax.experimental.pallas import tpu as pltpu
from jax.experimental.pallas import tpu_sc as plsc

D_MODEL = 64
CHUNK = 128
NBUF = 8
BV = 512


def _table_transpose(wt):
    v = wt.shape[1]

    def body(in_ref, out_ref):
        out_ref[...] = in_ref[...].T

    return pl.pallas_call(
        body,
        grid=(pl.cdiv(v, BV),),
        in_specs=[pl.BlockSpec((D_MODEL, BV), lambda i: (0, i))],
        out_specs=pl.BlockSpec((BV, D_MODEL), lambda i: (i, 0)),
        out_shape=jax.ShapeDtypeStruct((v, D_MODEL), jnp.float32),
    )(wt)


def _out_format(rows, n_t, n_b):
    cb_per_t = n_b // CHUNK

    def body(in_ref, out_ref):
        out_ref[0] = in_ref[...].T

    return pl.pallas_call(
        body,
        grid=(n_t, cb_per_t),
        in_specs=[pl.BlockSpec((CHUNK, D_MODEL), lambda t, c: (t * cb_per_t + c, 0))],
        out_specs=pl.BlockSpec((1, D_MODEL, CHUNK), lambda t, c: (t, 0, c)),
        out_shape=jax.ShapeDtypeStruct((n_t, D_MODEL, n_b), jnp.float32),
    )(rows)


@functools.partial(jax.jit, static_argnums=(2,))
def _sc_gather(idx2d, table, chunks_per_w):
    n_rows = idx2d.shape[0] * idx2d.shape[1]
    mesh = plsc.VectorSubcoreMesh(core_axis_name="c", subcore_axis_name="s")
    num_cores = mesh.num_cores
    n_groups = chunks_per_w // NBUF

    @functools.partial(
        pl.kernel,
        out_type=jax.ShapeDtypeStruct((n_rows, D_MODEL), jnp.float32),
        mesh=mesh,
        scratch_types=[
            pltpu.VMEM((chunks_per_w, CHUNK), jnp.int32),
            pltpu.VMEM((NBUF * CHUNK, D_MODEL), jnp.float32),
            pltpu.SemaphoreType.DMA,
        ],
        compiler_params=pltpu.CompilerParams(use_tc_tiling_on_sc=False),
    )
    def k(idx_hbm, table_hbm, out_hbm, idx_v, rows_v, sem):
        wid = lax.axis_index("s") * num_cores + lax.axis_index("c")
        chunk0 = wid * chunks_per_w
        pltpu.sync_copy(idx_hbm.at[pl.ds(chunk0, chunks_per_w)], idx_v)

        @pl.loop(0, n_groups)
        def _(g):
            descs = [
                pltpu.async_copy(
                    table_hbm.at[idx_v.at[g * NBUF + b]],
                    rows_v.at[pl.ds(b * CHUNK, CHUNK)],
                    sem,
                )
                for b in range(NBUF)
            ]
            for d in descs:
                d.wait()
            pltpu.sync_copy(
                rows_v, out_hbm.at[pl.ds((chunk0 + g * NBUF) * CHUNK, NBUF * CHUNK)]
            )

    return k(idx2d, table)


def kernel(tokens, W_E):
    b, t = tokens.shape
    n_rows = b * t
    num_workers = 32
    assert n_rows % (num_workers * CHUNK) == 0
    chunks_per_w = n_rows // (num_workers * CHUNK)
    idx2d = tokens.T.reshape(n_rows // CHUNK, CHUNK).astype(jnp.int32)
    table = _table_transpose(W_E.T)
    rows = _sc_gather(idx2d, table, chunks_per_w)
    out = _out_format(rows, t, b)
    return jnp.transpose(out, (2, 0, 1))


# speedup vs baseline: 2.7064x; 2.7064x over previous
import functools

import jax
import jax.numpy as jnp
from jax import lax
from jax.experimental import pallas as pl
from jax.experimental.pallas import tpu as pltpu
from jax.experimental.pallas import tpu_sc as plsc

D_MODEL = 64
CHUNK = 128
NBUF = 8
BV = 2048


def _eye():
    r = lax.broadcasted_iota(jnp.int32, (D_MODEL, D_MODEL), 0)
    c = lax.broadcasted_iota(jnp.int32, (D_MODEL, D_MODEL), 1)
    return jnp.where(r == c, 1.0, 0.0).astype(jnp.float32)


def _table_transpose(wt):
    v = wt.shape[1]

    def body(in_ref, out_ref):
        out_ref[...] = lax.dot_general(
            in_ref[...],
            _eye(),
            (((0,), (0,)), ((), ())),
            preferred_element_type=jnp.float32,
            precision=lax.Precision.HIGHEST,
        )

    return pl.pallas_call(
        body,
        grid=(pl.cdiv(v, BV),),
        in_specs=[pl.BlockSpec((D_MODEL, BV), lambda i: (0, i))],
        out_specs=pl.BlockSpec((BV, D_MODEL), lambda i: (i, 0)),
        out_shape=jax.ShapeDtypeStruct((v, D_MODEL), jnp.float32),
    )(wt)


def _out_format(rows, n_t, n_b):
    cb_per_t = n_b // BV

    def body(in_ref, out_ref):
        out_ref[0] = lax.dot_general(
            _eye(),
            in_ref[...],
            (((1,), (1,)), ((), ())),
            preferred_element_type=jnp.float32,
            precision=lax.Precision.HIGHEST,
        )

    return pl.pallas_call(
        body,
        grid=(n_t, cb_per_t),
        in_specs=[pl.BlockSpec((BV, D_MODEL), lambda t, c: (t * cb_per_t + c, 0))],
        out_specs=pl.BlockSpec((1, D_MODEL, BV), lambda t, c: (t, 0, c)),
        out_shape=jax.ShapeDtypeStruct((n_t, D_MODEL, n_b), jnp.float32),
    )(rows)


@functools.partial(jax.jit, static_argnums=(2,))
def _sc_gather(idx2d, table, chunks_per_w):
    n_rows = idx2d.shape[0] * idx2d.shape[1]
    mesh = plsc.VectorSubcoreMesh(core_axis_name="c", subcore_axis_name="s")
    num_cores = mesh.num_cores
    n_groups = chunks_per_w // NBUF

    @functools.partial(
        pl.kernel,
        out_type=jax.ShapeDtypeStruct((n_rows, D_MODEL), jnp.float32),
        mesh=mesh,
        scratch_types=[
            pltpu.VMEM((chunks_per_w, CHUNK), jnp.int32),
            pltpu.VMEM((NBUF * CHUNK, D_MODEL), jnp.float32),
            pltpu.SemaphoreType.DMA,
        ],
        compiler_params=pltpu.CompilerParams(use_tc_tiling_on_sc=False),
    )
    def k(idx_hbm, table_hbm, out_hbm, idx_v, rows_v, sem):
        wid = lax.axis_index("s") * num_cores + lax.axis_index("c")
        chunk0 = wid * chunks_per_w
        pltpu.sync_copy(idx_hbm.at[pl.ds(chunk0, chunks_per_w)], idx_v)

        @pl.loop(0, n_groups)
        def _(g):
            descs = [
                pltpu.async_copy(
                    table_hbm.at[idx_v.at[g * NBUF + b]],
                    rows_v.at[pl.ds(b * CHUNK, CHUNK)],
                    sem,
                )
                for b in range(NBUF)
            ]
            for d in descs:
                d.wait()
            pltpu.sync_copy(
                rows_v, out_hbm.at[pl.ds((chunk0 + g * NBUF) * CHUNK, NBUF * CHUNK)]
            )

    return k(idx2d, table)


def kernel(tokens, W_E):
    b, t = tokens.shape
    n_rows = b * t
    num_workers = 32
    assert n_rows % (num_workers * CHUNK) == 0
    chunks_per_w = n_rows // (num_workers * CHUNK)
    idx2d = tokens.T.reshape(n_rows // CHUNK, CHUNK).astype(jnp.int32)
    table = _table_transpose(W_E.T)
    rows = _sc_gather(idx2d, table, chunks_per_w)
    out = _out_format(rows, t, b)
    return jnp.transpose(out, (2, 0, 1))


# speedup vs baseline: 4.2783x; 1.5808x over previous
import functools

import jax
import jax.numpy as jnp
from jax import lax
from jax.experimental import pallas as pl
from jax.experimental.pallas import tpu as pltpu
from jax.experimental.pallas import tpu_sc as plsc

D_MODEL = 64
CHUNK = 128
NBUF = 8


@functools.partial(jax.jit, static_argnums=(2, 3))
def _embed_gather(idx2d, table, num_workers, chunks_per_w):
    n_rows = idx2d.shape[0] * idx2d.shape[1]
    mesh = plsc.VectorSubcoreMesh(core_axis_name="c", subcore_axis_name="s")
    num_cores = mesh.num_cores
    n_groups = chunks_per_w // NBUF

    @functools.partial(
        pl.kernel,
        out_type=jax.ShapeDtypeStruct((n_rows, D_MODEL), jnp.float32),
        mesh=mesh,
        scratch_types=[
            pltpu.VMEM((chunks_per_w, CHUNK), jnp.int32),
            pltpu.VMEM((NBUF * CHUNK, D_MODEL), jnp.float32),
            pltpu.SemaphoreType.DMA,
        ],
        compiler_params=pltpu.CompilerParams(use_tc_tiling_on_sc=False),
    )
    def k(idx_hbm, table_hbm, out_hbm, idx_v, rows_v, sem):
        wid = lax.axis_index("s") * num_cores + lax.axis_index("c")
        chunk0 = wid * chunks_per_w
        pltpu.sync_copy(idx_hbm.at[pl.ds(chunk0, chunks_per_w)], idx_v)

        @pl.loop(0, n_groups)
        def _(g):
            descs = [
                pltpu.async_copy(
                    table_hbm.at[idx_v.at[g * NBUF + b]],
                    rows_v.at[pl.ds(b * CHUNK, CHUNK)],
                    sem,
                )
                for b in range(NBUF)
            ]
            for d in descs:
                d.wait()
            pltpu.sync_copy(
                rows_v, out_hbm.at[pl.ds((chunk0 + g * NBUF) * CHUNK, NBUF * CHUNK)]
            )

    return k(idx2d, table)


def kernel(tokens, W_E):
    b, t = tokens.shape
    n_rows = b * t
    num_workers = 32
    assert n_rows % (num_workers * CHUNK) == 0
    chunks_per_w = n_rows // (num_workers * CHUNK)
    idx2d = tokens.reshape(n_rows // CHUNK, CHUNK).astype(jnp.int32)
    out = _embed_gather(idx2d, W_E, num_workers, chunks_per_w)
    return out.reshape(b, t, W_E.shape[1])
